# baseline (speedup 1.0000x reference)
import jax
import jax.numpy as jnp
from jax import lax
from jax.experimental import pallas as pl
from jax.experimental.pallas import tpu as pltpu
from jax.experimental.pallas import tpu_sc as plsc

_NF = 26
_NP = 325
_B = 1024
_D = 64
_NC = 2
_NS = 16
_NW = _NC * _NS
_R = _B // _NW
_L = 16
_G = 13
_NB = _NP // _G

_PAIRS = [(i, j) for i in range(_NF) for j in range(i + 1, _NF)]


def _sc_body(x_hbm, out_hbm, xv, ob0, ob1, sem):
    wid = lax.axis_index("s") * _NC + lax.axis_index("c")
    r0 = wid * _R
    pltpu.sync_copy(x_hbm.at[:, pl.ds(r0, _R), :], xv)

    bufs = (ob0, ob1)

    def compute_block(buf, block_pairs):
        def row_step(r, acc):
            for g, (i, j) in enumerate(block_pairs):
                for c in range(_D // _L):
                    sl = pl.ds(c * _L, _L)
                    buf[g, r, sl] = xv[i, r, sl] * xv[j, r, sl]
            return acc

        lax.fori_loop(0, _R, row_step, 0)

    for b in range(_NB):
        buf = bufs[b % 2]
        p0 = b * _G
        if b >= 2:
            pltpu.make_async_copy(
                buf,
                out_hbm.at[pl.ds((b - 2) * _G, _G), pl.ds(r0, _R), :],
                sem.at[b % 2],
            ).wait()
        compute_block(buf, _PAIRS[p0:p0 + _G])
        pltpu.async_copy(
            buf,
            out_hbm.at[pl.ds(p0, _G), pl.ds(r0, _R), :],
            sem.at[b % 2],
        )

    for b in (_NB - 2, _NB - 1):
        pltpu.make_async_copy(
            bufs[b % 2],
            out_hbm.at[pl.ds(b * _G, _G), pl.ds(r0, _R), :],
            sem.at[b % 2],
        ).wait()


def kernel(x):
    k = pl.kernel(
        _sc_body,
        out_type=jax.ShapeDtypeStruct((_NP, _B, _D), jnp.float32),
        mesh=plsc.VectorSubcoreMesh(core_axis_name="c", subcore_axis_name="s"),
        compiler_params=pltpu.CompilerParams(use_tc_tiling_on_sc=False),
        scratch_types=[
            pltpu.VMEM((_NF, _R, _D), jnp.float32),
            pltpu.VMEM((_G, _R, _D), jnp.float32),
            pltpu.VMEM((_G, _R, _D), jnp.float32),
            pltpu.SemaphoreType.DMA((2,)),
        ],
    )
    return k(x)
